# baseline (speedup 1.0000x reference)
import functools

import jax
import jax.numpy as jnp
from jax import lax
from jax.experimental import pallas as pl
from jax.experimental.pallas import tpu as pltpu
from jax.experimental.pallas import tpu_sc as plsc

N_ITEM = 10000
N_TAG = 2000
D = 128
NI = 10000
NT = 2000
E = 320000
DA = 144

NC = 2
NS = 16
NW = NC * NS
EW = E // NW


def _zero_vmem_2d(buf, rows, cols):
  ncol = cols // 16

  def body(i, _):
    r = i // ncol
    j = i % ncol
    buf[r, pl.ds(j * 16, 16)] = jnp.zeros((16,), jnp.float32)
    return 0

  lax.fori_loop(0, rows * ncol, body, 0)


def _make_sc_phase(n_seg, compose, n_ids, K, S, zr):
  mesh = plsc.VectorSubcoreMesh(core_axis_name="c", subcore_axis_name="s")
  nzc = n_seg // zr
  NCHUNK = EW // K
  S2 = 2 * S
  ROUNDS = (NCHUNK + S2 - 1) // S2

  out_type = [jax.ShapeDtypeStruct((NC, n_seg, DA), jnp.float32)]
  if compose:
    out_type.append(jax.ShapeDtypeStruct((E,), jnp.int32))

  scratch = (
      [pltpu.VMEM_SHARED((n_seg, DA), jnp.float32)]
      + [pltpu.VMEM((max(n_ids, 16),), jnp.int32)] * 2
      + [pltpu.VMEM((S, K, DA), jnp.float32),
         pltpu.VMEM((zr, DA), jnp.float32)]
      + [pltpu.VMEM((K,), jnp.int32)] * (2 * S2)
      + [pltpu.SemaphoreType.DMA] * (S2 + 2 * S)
  )

  @functools.partial(
      pl.kernel, mesh=mesh,
      out_type=tuple(out_type) if compose else out_type[0],
      scratch_types=scratch,
      compiler_params=pltpu.CompilerParams(use_tc_tiling_on_sc=False,
                                           needs_layout_passes=False),
  )
  def phase(gidx_hbm, sidx_hbm, ids_hbm, table_hbm, out_hbm, *rest):
    if compose:
      comp_hbm, acc, idsv, gall, rows, zbuf, *vs = rest
    else:
      acc, idsv, gall, rows, zbuf, *vs = rest
      comp_hbm = gidx_hbm
    gv = vs[:S2]
    sv = vs[S2:2 * S2]
    isem = vs[2 * S2:3 * S2]
    gsem = vs[3 * S2:3 * S2 + S]
    ssem = vs[3 * S2 + S:]
    c = lax.axis_index("c")
    s = lax.axis_index("s")
    wid = s * NC + c
    e0 = wid * EW

    _zero_vmem_2d(zbuf, zr, DA)

    def zbody(z, _):
      @pl.when(z % NS == s)
      def _():
        pltpu.sync_copy(zbuf, acc.at[pl.ds(z * zr, zr), :])
      return 0

    lax.fori_loop(0, nzc, zbody, 0)

    if compose:
      pltpu.sync_copy(ids_hbm, idsv)
      pltpu.sync_copy(gidx_hbm.at[pl.ds(e0, EW)], gall.at[pl.ds(0, EW)])

      def cbody(i, _):
        raw = gall[pl.ds(i * 16, 16)]
        gall[pl.ds(i * 16, 16)] = plsc.load_gather(idsv, [raw])
        return 0

      lax.fori_loop(0, EW // 16, cbody, 0)
      pltpu.sync_copy(gall.at[pl.ds(0, EW)], comp_hbm.at[pl.ds(e0, EW)])
    plsc.subcore_barrier()

    def issue_idx(y, g):
      pltpu.async_copy(comp_hbm.at[pl.ds(e0 + g * K, K)], gv[y], isem[y])
      pltpu.async_copy(sidx_hbm.at[pl.ds(e0 + g * K, K)], sv[y], isem[y])

    def wait_idx(y):
      pltpu.make_async_copy(comp_hbm.at[pl.ds(e0, K)], gv[y], isem[y]).wait()
      pltpu.make_async_copy(sidx_hbm.at[pl.ds(e0, K)], sv[y], isem[y]).wait()

    def wait_scatter(x, y):
      pltpu.make_async_copy(rows.at[x], acc.at[sv[y]], ssem[x]).wait()

    for y in range(S2):
      issue_idx(y, y)

    def body(q, _):
      for r in range(2):
        base = q * S2 + r * S
        for x in range(S):
          g = base + x
          y = r * S + x
          yp = (y - S) % S2

          @pl.when(g < NCHUNK)
          def _():
            wait_idx(y)

            @pl.when(g >= S)
            def _():
              wait_scatter(x, yp)

              @pl.when(g + S < NCHUNK)
              def _():
                issue_idx(yp, g + S)

            pltpu.async_copy(table_hbm.at[gv[y]], rows.at[x], gsem[x])
        for x in range(S):
          g = base + x
          y = r * S + x

          @pl.when(g < NCHUNK)
          def _():
            pltpu.make_async_copy(table_hbm.at[gv[y]],
                                  rows.at[x], gsem[x]).wait()
            pltpu.async_copy(rows.at[x], acc.at[sv[y]], ssem[x], add=True)
      return 0

    lax.fori_loop(0, ROUNDS, body, 0)
    for x in range(S):
      wait_scatter(x, x)
    plsc.subcore_barrier()

    def dbody(z, _):
      @pl.when(z % NS == s)
      def _():
        pltpu.sync_copy(acc.at[pl.ds(z * zr, zr), :],
                        out_hbm.at[c, pl.ds(z * zr, zr), :])
      return 0

    lax.fori_loop(0, nzc, dbody, 0)

  return phase


_phase_a = _make_sc_phase(NT, compose=True, n_ids=NI, K=80, S=7, zr=40)
_phase_b = _make_sc_phase(NI, compose=False, n_ids=16, K=40, S=6, zr=8)


def _combine_body(p_ref, o_ref):
  p = p_ref[0] + p_ref[1]
  cnt = jnp.clip(p[:, 128:129], 1.0, None)
  col = lax.broadcasted_iota(jnp.int32, (NT, DA), 1)
  o_ref[...] = jnp.where(col < D, p / cnt,
                         jnp.where(col == D, 1.0, 0.0))


_combine = pl.pallas_call(
    _combine_body,
    out_shape=jax.ShapeDtypeStruct((NT, DA), jnp.float32),
)

BR = 400


def _final_body(p_ref, ws_ref, wn_ref, b_ref, wf_ref, o_ref):
  p = p_ref[0] + p_ref[1]
  cnt = jnp.clip(p[:, 128:129], 1.0, None)
  h = p[:, :D] / cnt
  sage = jnp.dot(h, ws_ref[...] + wn_ref[...],
                 preferred_element_type=jnp.float32) + b_ref[...]
  o_ref[...] = jnp.tanh(jnp.dot(sage, wf_ref[...],
                                preferred_element_type=jnp.float32))


_final = pl.pallas_call(
    _final_body,
    grid=(NI // BR,),
    in_specs=[
        pl.BlockSpec((NC, BR, DA), lambda i: (0, i, 0)),
        pl.BlockSpec((D, D), lambda i: (0, 0)),
        pl.BlockSpec((D, D), lambda i: (0, 0)),
        pl.BlockSpec((1, D), lambda i: (0, 0)),
        pl.BlockSpec((D, N_ITEM), lambda i: (0, 0)),
    ],
    out_specs=pl.BlockSpec((BR, N_ITEM), lambda i: (i, 0)),
    out_shape=jax.ShapeDtypeStruct((NI, N_ITEM), jnp.float32),
)


def kernel(item_ids, tag_ids, edge_src, edge_dst, item_emb, tag_emb,
           W_self, W_neigh, b_neigh, W_final):
  del tag_ids, tag_emb
  item_ids = item_ids.astype(jnp.int32)
  edge_src = edge_src.astype(jnp.int32)
  edge_dst = edge_dst.astype(jnp.int32)
  item_emb_aug = jnp.concatenate(
      [item_emb,
       jnp.ones((N_ITEM, 1), jnp.float32),
       jnp.zeros((N_ITEM, DA - D - 1), jnp.float32)], axis=1)

  tag_part, _ = _phase_a(edge_src, edge_dst, item_ids, item_emb_aug)
  h_tag_aug = _combine(tag_part)
  zero_ids = jnp.zeros((16,), jnp.int32)
  item_part = _phase_b(edge_dst, edge_src, zero_ids, h_tag_aug)
  return _final(item_part, W_self, W_neigh, b_neigh.reshape(1, D), W_final)


# speedup vs baseline: 1.1007x; 1.1007x over previous
import functools

import jax
import jax.numpy as jnp
from jax import lax
from jax.experimental import pallas as pl
from jax.experimental.pallas import tpu as pltpu
from jax.experimental.pallas import tpu_sc as plsc

N_ITEM = 10000
N_TAG = 2000
D = 128
NI = 10000
NT = 2000
E = 320000
DA = 144

NC = 2
NS = 16
NW = NC * NS
EW = E // NW


def _zero_vmem_2d(buf, rows, cols):
  ncol = cols // 16

  def body(i, _):
    r = i // ncol
    j = i % ncol
    buf[r, pl.ds(j * 16, 16)] = jnp.zeros((16,), jnp.float32)
    return 0

  lax.fori_loop(0, rows * ncol, body, 0)


def _make_sc_phase(n_seg, compose, n_ids, K, S, zr):
  mesh = plsc.VectorSubcoreMesh(core_axis_name="c", subcore_axis_name="s")
  nzc = n_seg // zr
  NCHUNK = EW // K
  S2 = 2 * S
  ROUNDS = (NCHUNK + S2 - 1) // S2

  out_type = [jax.ShapeDtypeStruct((NC, n_seg, DA), jnp.float32)]
  if compose:
    out_type.append(jax.ShapeDtypeStruct((E,), jnp.int32))

  scratch = (
      [pltpu.VMEM_SHARED((n_seg, DA), jnp.float32)]
      + [pltpu.VMEM((max(n_ids, 16),), jnp.int32)] * 2
      + [pltpu.VMEM((S, K, DA), jnp.float32),
         pltpu.VMEM((zr, DA), jnp.float32)]
      + [pltpu.VMEM((K,), jnp.int32)] * (2 * S2)
      + [pltpu.SemaphoreType.DMA] * (S2 + 2 * S)
  )

  @functools.partial(
      pl.kernel, mesh=mesh,
      out_type=tuple(out_type) if compose else out_type[0],
      scratch_types=scratch,
      compiler_params=pltpu.CompilerParams(use_tc_tiling_on_sc=False,
                                           needs_layout_passes=False),
  )
  def phase(gidx_hbm, sidx_hbm, ids_hbm, table_hbm, out_hbm, *rest):
    if compose:
      comp_hbm, acc, idsv, gall, rows, zbuf, *vs = rest
    else:
      acc, idsv, gall, rows, zbuf, *vs = rest
      comp_hbm = gidx_hbm
    gv = vs[:S2]
    sv = vs[S2:2 * S2]
    isem = vs[2 * S2:3 * S2]
    gsem = vs[3 * S2:3 * S2 + S]
    ssem = vs[3 * S2 + S:]
    c = lax.axis_index("c")
    s = lax.axis_index("s")
    wid = s * NC + c
    e0 = wid * EW

    _zero_vmem_2d(zbuf, zr, DA)

    def zbody(z, _):
      @pl.when(z % NS == s)
      def _():
        pltpu.sync_copy(zbuf, acc.at[pl.ds(z * zr, zr), :])
      return 0

    lax.fori_loop(0, nzc, zbody, 0)

    if compose:
      pltpu.sync_copy(ids_hbm, idsv)
      pltpu.sync_copy(gidx_hbm.at[pl.ds(e0, EW)], gall.at[pl.ds(0, EW)])

      def cbody(i, _):
        raw = gall[pl.ds(i * 16, 16)]
        gall[pl.ds(i * 16, 16)] = plsc.load_gather(idsv, [raw])
        return 0

      lax.fori_loop(0, EW // 16, cbody, 0)
      pltpu.sync_copy(gall.at[pl.ds(0, EW)], comp_hbm.at[pl.ds(e0, EW)])
    plsc.subcore_barrier()

    def issue_idx(y, g):
      pltpu.async_copy(comp_hbm.at[pl.ds(e0 + g * K, K)], gv[y], isem[y])
      pltpu.async_copy(sidx_hbm.at[pl.ds(e0 + g * K, K)], sv[y], isem[y])

    def wait_idx(y):
      pltpu.make_async_copy(comp_hbm.at[pl.ds(e0, K)], gv[y], isem[y]).wait()
      pltpu.make_async_copy(sidx_hbm.at[pl.ds(e0, K)], sv[y], isem[y]).wait()

    def wait_scatter(x, y):
      pltpu.make_async_copy(rows.at[x], acc.at[sv[y]], ssem[x]).wait()

    for y in range(S2):
      issue_idx(y, y)

    def body(q, _):
      for r in range(2):
        base = q * S2 + r * S
        for x in range(S):
          g = base + x
          y = r * S + x
          yp = (y - S) % S2

          @pl.when(g < NCHUNK)
          def _():
            wait_idx(y)

            @pl.when(g >= S)
            def _():
              wait_scatter(x, yp)

              @pl.when(g + S < NCHUNK)
              def _():
                issue_idx(yp, g + S)

            pltpu.async_copy(table_hbm.at[gv[y]], rows.at[x], gsem[x])
        for x in range(S):
          g = base + x
          y = r * S + x

          @pl.when(g < NCHUNK)
          def _():
            pltpu.make_async_copy(table_hbm.at[gv[y]],
                                  rows.at[x], gsem[x]).wait()
            pltpu.async_copy(rows.at[x], acc.at[sv[y]], ssem[x], add=True)
      return 0

    lax.fori_loop(0, ROUNDS, body, 0)
    for x in range(S):
      wait_scatter(x, x)
    plsc.subcore_barrier()

    def dbody(z, _):
      @pl.when(z % NS == s)
      def _():
        pltpu.sync_copy(acc.at[pl.ds(z * zr, zr), :],
                        out_hbm.at[c, pl.ds(z * zr, zr), :])
      return 0

    lax.fori_loop(0, nzc, dbody, 0)

  return phase


_phase_a = _make_sc_phase(NT, compose=True, n_ids=NI, K=80, S=7, zr=40)
_phase_b = _make_sc_phase(NI, compose=False, n_ids=16, K=40, S=5, zr=40)


def _combine_body(p_ref, o_ref):
  p = p_ref[0] + p_ref[1]
  cnt = jnp.clip(p[:, 128:129], 1.0, None)
  col = lax.broadcasted_iota(jnp.int32, (NT, DA), 1)
  o_ref[...] = jnp.where(col < D, p / cnt,
                         jnp.where(col == D, 1.0, 0.0))


_combine = pl.pallas_call(
    _combine_body,
    out_shape=jax.ShapeDtypeStruct((NT, DA), jnp.float32),
)

BR = 400


def _final_body(p_ref, ws_ref, wn_ref, b_ref, wf_ref, o_ref):
  p = p_ref[0] + p_ref[1]
  cnt = jnp.clip(p[:, 128:129], 1.0, None)
  h = p[:, :D] / cnt
  sage = jnp.dot(h, ws_ref[...] + wn_ref[...],
                 preferred_element_type=jnp.float32) + b_ref[...]
  o_ref[...] = jnp.tanh(jnp.dot(sage, wf_ref[...],
                                preferred_element_type=jnp.float32))


_final = pl.pallas_call(
    _final_body,
    grid=(NI // BR,),
    in_specs=[
        pl.BlockSpec((NC, BR, DA), lambda i: (0, i, 0)),
        pl.BlockSpec((D, D), lambda i: (0, 0)),
        pl.BlockSpec((D, D), lambda i: (0, 0)),
        pl.BlockSpec((1, D), lambda i: (0, 0)),
        pl.BlockSpec((D, N_ITEM), lambda i: (0, 0)),
    ],
    out_specs=pl.BlockSpec((BR, N_ITEM), lambda i: (i, 0)),
    out_shape=jax.ShapeDtypeStruct((NI, N_ITEM), jnp.float32),
)


def kernel(item_ids, tag_ids, edge_src, edge_dst, item_emb, tag_emb,
           W_self, W_neigh, b_neigh, W_final):
  del tag_ids, tag_emb
  item_ids = item_ids.astype(jnp.int32)
  edge_src = edge_src.astype(jnp.int32)
  edge_dst = edge_dst.astype(jnp.int32)
  item_emb_aug = jnp.concatenate(
      [item_emb,
       jnp.ones((N_ITEM, 1), jnp.float32),
       jnp.zeros((N_ITEM, DA - D - 1), jnp.float32)], axis=1)

  tag_part, _ = _phase_a(edge_src, edge_dst, item_ids, item_emb_aug)
  h_tag_aug = _combine(tag_part)
  zero_ids = jnp.zeros((16,), jnp.int32)
  item_part = _phase_b(edge_dst, edge_src, zero_ids, h_tag_aug)
  return _final(item_part, W_self, W_neigh, b_neigh.reshape(1, D), W_final)
